# baseline (speedup 1.0000x reference)
import jax
import jax.numpy as jnp
from jax import lax
from jax.experimental import pallas as pl
from jax.experimental.pallas import tpu as pltpu

_N = 32768
_DIM = 1024
_HID = 128
_K = 32
_BLK = 1024
_GRID = _N // _BLK
_SR = _N // 128
_BR = _BLK // 128

_NEG = float('-inf')


def _body(x_blk, w1, b1, w2row, b2, x_any, out_ref,
          sc_ref, candv_ref, candr_ref, rows_ref, idx_ref, sem):
    i = pl.program_id(0)
    h = jnp.maximum(
        jnp.dot(x_blk[...], w1[...], preferred_element_type=jnp.float32)
        + b1[...], 0.0)
    s = jnp.sum(h * w2row[...], axis=1) + b2[0, 0]
    sc_ref[pl.ds(i * _BR, _BR), :] = s.reshape(_BR, 128)

    @pl.when(i == _GRID - 1)
    def _finalize():
        riota = lax.broadcasted_iota(jnp.int32, (_SR, 128), 0)
        scv = sc_ref[...]
        for t in range(_K):
            m = jnp.max(scv, axis=0, keepdims=True)
            r = jnp.min(jnp.where(scv == m, riota, jnp.int32(_SR)),
                        axis=0, keepdims=True)
            candv_ref[pl.ds(t, 1), :] = m
            candr_ref[pl.ds(t, 1), :] = r
            scv = jnp.where(riota == r, _NEG, scv)
        ciota = lax.broadcasted_iota(jnp.int32, (_K, 128), 1)
        lane1 = lax.broadcasted_iota(jnp.int32, (1, 128), 1)
        remv = candv_ref[...]
        remi = (candr_ref[...] * 128 + ciota).astype(jnp.float32)
        fillv = jnp.full((1, 128), _NEG, jnp.float32)
        filli = jnp.full((1, 128), float(_N), jnp.float32)
        bigi = jnp.float32(_N)
        idxacc = jnp.zeros((1, 128), jnp.float32)
        for t in range(_K):
            f = remv[0:1, :]
            fi = remi[0:1, :]
            m = jnp.max(f, axis=1, keepdims=True)
            eqm = f == m
            sel = jnp.min(jnp.where(eqm, fi, bigi),
                          axis=1, keepdims=True)
            idxacc = jnp.where(lane1 == t, sel, idxacc)
            adv = eqm & (fi == sel)
            remv_s = jnp.concatenate([remv[1:, :], fillv], axis=0)
            remi_s = jnp.concatenate([remi[1:, :], filli], axis=0)
            remv = jnp.where(adv, remv_s, remv)
            remi = jnp.where(adv, remi_s, remi)
        idxi = idxacc.astype(jnp.int32)
        for t in range(_K):
            idx_ref[t] = idxi[0, t]
        copies = []
        for t in range(_K):
            cp = pltpu.make_async_copy(
                x_any.at[pl.ds(idx_ref[t], 1), :],
                rows_ref.at[pl.ds(t, 1), :], sem)
            cp.start()
            copies.append(cp)
        for cp in copies:
            cp.wait()
        out_ref[...] = jnp.sum(rows_ref[...], axis=0,
                               keepdims=True) * (1.0 / _K)


def kernel(x, W1, b1, W2, b2):
    out = pl.pallas_call(
        _body,
        grid=(_GRID,),
        in_specs=[
            pl.BlockSpec((_BLK, _DIM), lambda i: (i, 0)),
            pl.BlockSpec((_DIM, _HID), lambda i: (0, 0)),
            pl.BlockSpec((1, _HID), lambda i: (0, 0)),
            pl.BlockSpec((1, _HID), lambda i: (0, 0)),
            pl.BlockSpec((1, 1), lambda i: (0, 0)),
            pl.BlockSpec(memory_space=pl.MemorySpace.ANY),
        ],
        out_specs=pl.BlockSpec((1, _DIM), lambda i: (0, 0)),
        out_shape=jax.ShapeDtypeStruct((1, _DIM), jnp.float32),
        scratch_shapes=[
            pltpu.VMEM((_SR, 128), jnp.float32),
            pltpu.VMEM((_K, 128), jnp.float32),
            pltpu.VMEM((_K, 128), jnp.int32),
            pltpu.VMEM((_K, _DIM), jnp.float32),
            pltpu.SMEM((_K,), jnp.int32),
            pltpu.SemaphoreType.DMA,
        ],
        compiler_params=pltpu.CompilerParams(
            dimension_semantics=("arbitrary",),
        ),
    )(x, W1, b1.reshape(1, _HID), W2.reshape(1, _HID),
      b2.reshape(1, 1), x)
    return out.reshape(_DIM)


# speedup vs baseline: 1.1979x; 1.1979x over previous
import jax
import jax.numpy as jnp
from jax import lax
from jax.experimental import pallas as pl
from jax.experimental.pallas import tpu as pltpu

_N = 32768
_DIM = 1024
_HID = 128
_K = 32
_BLK = 2048
_GRID = _N // _BLK
_SR = _N // 128
_BR = _BLK // 128

_NEG = float('-inf')


def _body(xa_blk, xb_blk, w1, b1, w2row, b2, x_any, out_ref,
          sc_ref, candv_ref, candr_ref, rows_ref, idx_ref, sem):
    i = pl.program_id(0)
    xcat = jnp.concatenate([xa_blk[...], xb_blk[...]], axis=0)
    h = jnp.maximum(
        jnp.dot(xcat, w1[...], preferred_element_type=jnp.float32)
        + b1[...], 0.0)
    s = jnp.sum(h * w2row[...], axis=1) + b2[0, 0]
    sh = _BR // 2
    s2d = s.reshape(_BR, 128)
    sc_ref[pl.ds(i * sh, sh), :] = s2d[:sh, :]
    sc_ref[pl.ds(_SR // 2 + i * sh, sh), :] = s2d[sh:, :]

    @pl.when(i == _GRID - 1)
    def _finalize():
        riota = lax.broadcasted_iota(jnp.int32, (_SR, 128), 0)
        scv = sc_ref[...]
        for t in range(_K):
            m = jnp.max(scv, axis=0, keepdims=True)
            r = jnp.min(jnp.where(scv == m, riota, jnp.int32(_SR)),
                        axis=0, keepdims=True)
            candv_ref[pl.ds(t, 1), :] = m
            candr_ref[pl.ds(t, 1), :] = r
            scv = jnp.where(riota == r, _NEG, scv)
        ciota = lax.broadcasted_iota(jnp.int32, (_K, 128), 1)
        lane1 = lax.broadcasted_iota(jnp.int32, (1, 128), 1)
        remv = candv_ref[...]
        remi = (candr_ref[...] * 128 + ciota).astype(jnp.float32)
        fillv = jnp.full((1, 128), _NEG, jnp.float32)
        filli = jnp.full((1, 128), float(_N), jnp.float32)
        bigi = jnp.float32(_N)
        idxacc = jnp.zeros((1, 128), jnp.float32)
        for t in range(_K):
            f = remv[0:1, :]
            fi = remi[0:1, :]
            m = jnp.max(f, axis=1, keepdims=True)
            eqm = f == m
            sel = jnp.min(jnp.where(eqm, fi, bigi),
                          axis=1, keepdims=True)
            idxacc = jnp.where(lane1 == t, sel, idxacc)
            adv = eqm & (fi == sel)
            remv_s = jnp.concatenate([remv[1:, :], fillv], axis=0)
            remi_s = jnp.concatenate([remi[1:, :], filli], axis=0)
            remv = jnp.where(adv, remv_s, remv)
            remi = jnp.where(adv, remi_s, remi)
        idxi = idxacc.astype(jnp.int32)
        for t in range(_K):
            idx_ref[t] = idxi[0, t]
        copies = []
        for t in range(_K):
            cp = pltpu.make_async_copy(
                x_any.at[pl.ds(idx_ref[t], 1), :],
                rows_ref.at[pl.ds(t, 1), :], sem)
            cp.start()
            copies.append(cp)
        for cp in copies:
            cp.wait()
        out_ref[...] = jnp.sum(rows_ref[...], axis=0,
                               keepdims=True) * (1.0 / _K)


def kernel(x, W1, b1, W2, b2):
    out = pl.pallas_call(
        _body,
        grid=(_GRID,),
        in_specs=[
            pl.BlockSpec((_BLK // 2, _DIM), lambda i: (i, 0)),
            pl.BlockSpec((_BLK // 2, _DIM), lambda i: (i + _GRID, 0)),
            pl.BlockSpec((_DIM, _HID), lambda i: (0, 0)),
            pl.BlockSpec((1, _HID), lambda i: (0, 0)),
            pl.BlockSpec((1, _HID), lambda i: (0, 0)),
            pl.BlockSpec((1, 1), lambda i: (0, 0)),
            pl.BlockSpec(memory_space=pl.MemorySpace.ANY),
        ],
        out_specs=pl.BlockSpec((1, _DIM), lambda i: (0, 0)),
        out_shape=jax.ShapeDtypeStruct((1, _DIM), jnp.float32),
        scratch_shapes=[
            pltpu.VMEM((_SR, 128), jnp.float32),
            pltpu.VMEM((_K, 128), jnp.float32),
            pltpu.VMEM((_K, 128), jnp.int32),
            pltpu.VMEM((_K, _DIM), jnp.float32),
            pltpu.SMEM((_K,), jnp.int32),
            pltpu.SemaphoreType.DMA,
        ],
        compiler_params=pltpu.CompilerParams(
            dimension_semantics=("arbitrary",),
        ),
    )(x, x, W1, b1.reshape(1, _HID), W2.reshape(1, _HID),
      b2.reshape(1, 1), x)
    return out.reshape(_DIM)
